# baseline (speedup 1.0000x reference)
import functools

import jax
import jax.numpy as jnp
from jax import lax
from jax.experimental import pallas as pl
from jax.experimental.pallas import tpu as pltpu
from jax.experimental.pallas import tpu_sc as plsc

_CH = 128
_NW = 32
_L = 16


@jax.jit
def kernel(input_ids, weight):
    B, T = input_ids.shape
    V, D = weight.shape
    n = B * T
    n_chunks = n // _CH
    n_grp = n_chunks // _NW
    assert n % _CH == 0 and n_chunks % _NW == 0
    assert n_grp % 8 == 0

    ids = input_ids.T.reshape(n_chunks, _CH).astype(jnp.int32)
    w2 = weight.reshape(V // 2, 2 * D)

    mesh = plsc.VectorSubcoreMesh(core_axis_name="c", subcore_axis_name="s")

    @functools.partial(
        pl.kernel,
        mesh=mesh,
        compiler_params=pltpu.CompilerParams(
            use_tc_tiling_on_sc=True, needs_layout_passes=False
        ),
        out_type=jax.ShapeDtypeStruct((n_chunks, _CH, D), jnp.float32),
        scratch_types=[
            pltpu.VMEM((n_grp, _CH), jnp.int32),
            pltpu.VMEM((2, _CH), jnp.int32),
            pltpu.VMEM((_CH, 2 * D), jnp.float32),
            pltpu.VMEM((_CH, 2 * D), jnp.float32),
            pltpu.VMEM((_CH, D), jnp.float32),
            pltpu.VMEM((_CH, D), jnp.float32),
            pltpu.SemaphoreType.DMA,
            pltpu.SemaphoreType.DMA,
            pltpu.SemaphoreType.DMA,
            pltpu.SemaphoreType.DMA,
        ],
    )
    def emb(ids_hbm, w_hbm, out_hbm, idx_v, pidx, sa0, sa1, sb0, sb1,
            g0, g1, o0, o1):
        wid = lax.axis_index("s") * 2 + lax.axis_index("c")
        cbase = wid * n_grp
        slaba = (sa0, sa1)
        slabb = (sb0, sb1)
        gsem = (g0, g1)
        osem = (o0, o1)

        pltpu.sync_copy(ids_hbm.at[pl.ds(cbase, n_grp)], idx_v)
        lanes = lax.iota(jnp.int32, _L)

        def prep_fire(grp, p):
            for k in range(_CH // _L):
                v = idx_v[grp, pl.ds(k * _L, _L)]
                pidx[p, pl.ds(k * _L, _L)] = lax.shift_right_logical(v, 1)
            pltpu.async_copy(w_hbm.at[pidx.at[p]], slaba[p], gsem[p])

        def step(grp, p):
            pltpu.make_async_copy(
                w_hbm.at[pidx.at[p]], slaba[p], gsem[p]
            ).wait()

            @pl.when(grp >= 2)
            def _():
                pltpu.make_async_copy(
                    slabb[p], out_hbm.at[cbase + grp - 2], osem[p]
                ).wait()

            for k in range(_CH // _L):
                ids16 = idx_v[grp, pl.ds(k * _L, _L)]
                col0 = lax.shift_left(
                    lax.bitwise_and(ids16, jnp.full((_L,), 1, jnp.int32)),
                    jnp.full((_L,), 6, jnp.int32),
                )
                row = lanes + jnp.int32(k * _L)

                def cbody(c, carry):
                    vals = plsc.load_gather(slaba[p], [row, col0 + c])
                    plsc.store_scatter(
                        slabb[p], [row, lax.broadcast(c, (_L,))], vals
                    )
                    return carry

                lax.fori_loop(0, D, cbody, 0, unroll=8)

            pltpu.async_copy(slabb[p], out_hbm.at[cbase + grp], osem[p])

            @pl.when(grp < n_grp - 2)
            def _():
                prep_fire(grp + 2, p)

        prep_fire(0, 0)
        prep_fire(1, 1)

        def body(g2, carry):
            for p in range(2):
                step(g2 * 2 + p, p)
            return carry

        lax.fori_loop(0, n_grp // 2, body, 0)

        pltpu.make_async_copy(
            slabb[0], out_hbm.at[cbase + n_grp - 2], osem[0]
        ).wait()
        pltpu.make_async_copy(
            slabb[1], out_hbm.at[cbase + n_grp - 1], osem[1]
        ).wait()

    out = emb(ids, w2)
    return out.reshape(T, B, D).transpose(1, 0, 2)


# speedup vs baseline: 2.1776x; 2.1776x over previous
import functools

import jax
import jax.numpy as jnp
from jax import lax
from jax.experimental import pallas as pl
from jax.experimental.pallas import tpu as pltpu
from jax.experimental.pallas import tpu_sc as plsc

_CH = 512
_NW = 32


@jax.jit
def kernel(input_ids, weight):
    B, T = input_ids.shape
    V, D = weight.shape
    n = B * T
    n_chunks = n // _CH
    n_grp = n_chunks // _NW
    assert n % _CH == 0 and n_chunks % _NW == 0 and n_grp % 2 == 0

    ids = input_ids.T.reshape(n_chunks, _CH).astype(jnp.int32)

    mesh = plsc.VectorSubcoreMesh(core_axis_name="c", subcore_axis_name="s")

    @functools.partial(
        pl.kernel,
        mesh=mesh,
        compiler_params=pltpu.CompilerParams(use_tc_tiling_on_sc=False),
        out_type=jax.ShapeDtypeStruct((n_chunks, _CH, D), jnp.float32),
        scratch_types=[
            pltpu.VMEM((n_grp, _CH), jnp.int32),
            pltpu.VMEM((_CH, D), jnp.float32),
            pltpu.VMEM((_CH, D), jnp.float32),
            pltpu.SemaphoreType.DMA,
            pltpu.SemaphoreType.DMA,
            pltpu.SemaphoreType.DMA,
            pltpu.SemaphoreType.DMA,
        ],
    )
    def emb(ids_hbm, w_hbm, out_hbm, idx_v, slab0, slab1, g0, g1, o0, o1):
        wid = lax.axis_index("s") * 2 + lax.axis_index("c")
        cbase = wid * n_grp
        slabs = (slab0, slab1)
        gsem = (g0, g1)
        osem = (o0, o1)

        pltpu.sync_copy(ids_hbm.at[pl.ds(cbase, n_grp)], idx_v)

        def fire(grp, p):
            pltpu.async_copy(w_hbm.at[idx_v.at[grp]], slabs[p], gsem[p])

        def drain_and_out(grp, p):
            pltpu.make_async_copy(
                w_hbm.at[idx_v.at[grp]], slabs[p], gsem[p]
            ).wait()
            return pltpu.async_copy(slabs[p], out_hbm.at[cbase + grp], osem[p])

        fire(0, 0)
        fire(1, 1)

        def body(g2, carry):
            for p in range(2):
                grp = g2 * 2 + p
                out_cp = drain_and_out(grp, p)
                out_cp.wait()
                fire(grp + 2, p)
            return carry

        lax.fori_loop(0, (n_grp - 2) // 2, body, 0)

        drain_and_out(n_grp - 2, 0).wait()
        drain_and_out(n_grp - 1, 1).wait()

    out = emb(ids, weight)
    return out.reshape(T, B, D).transpose(1, 0, 2)
